# baseline (speedup 1.0000x reference)
import functools

import jax
import jax.numpy as jnp
from jax import lax
from jax.experimental import pallas as pl
from jax.experimental.pallas import tpu as pltpu
from jax.experimental.pallas import tpu_sc as plsc

B = 256
T = 196
D = 768
P = 8192
LP = 5
K = 4
TD = T * D
LD = LP * D
OD = (T + LP) * D
PB = 1024
NEG = float("-inf")


def _copy_keys_body(x_ref, out_ref, xn_ref):
    c = pl.program_id(0)
    blk = x_ref[...]
    out_ref[...] = blk

    @pl.when(c == 0)
    def _():
        xn_ref[...] = blk

    @pl.when(c > 0)
    def _():
        xn_ref[...] += blk

    @pl.when(c == T - 1)
    def _():
        m = xn_ref[...] * jnp.float32(1.0 / T)
        sq = jnp.sum(m * m, axis=1, keepdims=True)
        xn_ref[...] = m * lax.rsqrt(jnp.maximum(sq, 1e-12))


def _copy_and_keys(x2):
    return pl.pallas_call(
        _copy_keys_body,
        grid=(T,),
        in_specs=[pl.BlockSpec((B, D), lambda c: (0, c))],
        out_specs=[
            pl.BlockSpec((B, D), lambda c: (0, c + LP)),
            pl.BlockSpec((B, D), lambda c: (0, 0)),
        ],
        out_shape=[
            jax.ShapeDtypeStruct((B, OD), jnp.float32),
            jax.ShapeDtypeStruct((B, D), jnp.float32),
        ],
    )(x2)


def _top4(s, iota, gbase):
    vs, gs = [], []
    for _ in range(K):
        v = jnp.max(s, axis=1, keepdims=True)
        a = jnp.min(jnp.where(s == v, iota, jnp.int32(2 ** 30)),
                    axis=1, keepdims=True)
        vs.append(v)
        gs.append(a + gbase)
        s = jnp.where(iota == a, NEG, s)
    return jnp.concatenate(vs, axis=1), jnp.concatenate(gs, axis=1)


def _sim_topk_body(xn_ref, pk_ref, sim_ref, tv_ref, ti_ref, idx_ref,
                   rv_ref, ri_ref):
    j = pl.program_id(0)
    xn = xn_ref[...]
    pk = pk_ref[...]
    sq = jnp.sum(pk * pk, axis=1, keepdims=True)
    pkn = pk * lax.rsqrt(jnp.maximum(sq, 1e-12))
    s = lax.dot_general(xn, pkn, (((1,), (1,)), ((), ())),
                        preferred_element_type=jnp.float32)
    sim_ref[...] = s

    iota = lax.broadcasted_iota(jnp.int32, (B, PB), 1)
    bv, bi = _top4(s, iota, j * PB)

    @pl.when(j == 0)
    def _():
        rv_ref[...] = bv
        ri_ref[...] = bi

    @pl.when(j > 0)
    def _():
        cv = jnp.concatenate([rv_ref[...], bv], axis=1)
        ci = jnp.concatenate([ri_ref[...], bi], axis=1)
        iota8 = lax.broadcasted_iota(jnp.int32, (B, 2 * K), 1)
        nvs, nis = [], []
        for _ in range(K):
            v = jnp.max(cv, axis=1, keepdims=True)
            a = jnp.min(jnp.where(cv == v, iota8, jnp.int32(2 ** 30)),
                        axis=1, keepdims=True)
            pick = jnp.sum(jnp.where(iota8 == a, ci, 0), axis=1,
                           keepdims=True)
            nvs.append(v)
            nis.append(pick)
            cv = jnp.where(iota8 == a, NEG, cv)
        rv_ref[...] = jnp.concatenate(nvs, axis=1)
        ri_ref[...] = jnp.concatenate(nis, axis=1)

    @pl.when(j == P // PB - 1)
    def _():
        tv_ref[...] = rv_ref[...]
        ti_ref[...] = ri_ref[...]
        idx_ref[...] = ri_ref[:, 0:1]


def _sim_topk(xn, prompt_key):
    return pl.pallas_call(
        _sim_topk_body,
        grid=(P // PB,),
        in_specs=[
            pl.BlockSpec((B, D), lambda j: (0, 0)),
            pl.BlockSpec((PB, D), lambda j: (j, 0)),
        ],
        out_specs=[
            pl.BlockSpec((B, PB), lambda j: (0, j)),
            pl.BlockSpec((B, K), lambda j: (0, 0)),
            pl.BlockSpec((B, K), lambda j: (0, 0)),
            pl.BlockSpec((B, 1), lambda j: (0, 0)),
        ],
        out_shape=[
            jax.ShapeDtypeStruct((B, P), jnp.float32),
            jax.ShapeDtypeStruct((B, K), jnp.float32),
            jax.ShapeDtypeStruct((B, K), jnp.int32),
            jax.ShapeDtypeStruct((B, 1), jnp.int32),
        ],
        scratch_shapes=[
            pltpu.VMEM((B, K), jnp.float32),
            pltpu.VMEM((B, K), jnp.int32),
        ],
    )(xn, prompt_key)


def _gather_rows(p2, idx):
    info = plsc.get_sparse_core_info()
    nw = info.num_cores * info.num_subcores
    bpw = B // nw
    mesh = plsc.VectorSubcoreMesh(core_axis_name="c", subcore_axis_name="s")

    @functools.partial(
        pl.kernel,
        out_type=jax.ShapeDtypeStruct((B, LD), jnp.float32),
        mesh=mesh,
        scratch_types=[
            pltpu.VMEM((bpw,), jnp.int32),
            pltpu.VMEM((bpw, LD), jnp.float32),
            pltpu.SemaphoreType.DMA,
        ],
    )
    def gath(p_hbm, idx_hbm, out_hbm, idx_v, rows_v, sem):
        wid = lax.axis_index("s") * info.num_cores + lax.axis_index("c")
        base = wid * bpw
        pltpu.sync_copy(idx_hbm.at[pl.ds(base, bpw)], idx_v)
        pltpu.async_copy(p_hbm.at[idx_v], rows_v, sem).wait()
        pltpu.sync_copy(rows_v, out_hbm.at[pl.ds(base, bpw)])

    return gath(p2, idx)


def _insert_body(g_ref, big_ref, out_ref):
    del big_ref
    out_ref[...] = g_ref[...]


def _insert(g2, out2):
    return pl.pallas_call(
        _insert_body,
        grid=(LP,),
        in_specs=[
            pl.BlockSpec((B, D), lambda j: (0, j)),
            pl.BlockSpec(memory_space=pl.ANY),
        ],
        out_specs=pl.BlockSpec((B, D), lambda j: (0, j)),
        out_shape=jax.ShapeDtypeStruct((B, OD), jnp.float32),
        input_output_aliases={1: 0},
    )(g2, out2)


def kernel(x_embed, prompt, prompt_key):
    x2 = x_embed.reshape(B, TD)
    p2 = prompt.reshape(P, LD)
    out2, xn = _copy_and_keys(x2)
    sim, top_k_sim, top_k_idx, idxc = _sim_topk(xn, prompt_key)
    idx = idxc.reshape(B)
    g2 = _gather_rows(p2, idx)
    outf = _insert(g2, out2)
    return (sim, top_k_sim, top_k_idx, idx, outf.reshape(B, T + LP, D))


# speedup vs baseline: 1.3892x; 1.3892x over previous
import functools

import jax
import jax.numpy as jnp
from jax import lax
from jax.experimental import pallas as pl
from jax.experimental.pallas import tpu as pltpu
from jax.experimental.pallas import tpu_sc as plsc

B = 256
T = 196
D = 768
P = 8192
LP = 5
K = 4
PB = 1024
BB = 8
NEG = float("-inf")


def _copy_keys_body(x_ref, out_ref, xn_ref):
    x = x_ref[...]
    out_ref[:, LP + 3:, :] = x[:, 3:, :]
    m = jnp.sum(x, axis=1) * jnp.float32(1.0 / T)
    sq = jnp.sum(m * m, axis=1, keepdims=True)
    xn_ref[...] = m * lax.rsqrt(jnp.maximum(sq, 1e-12))


def _copy_and_keys(x_embed):
    return pl.pallas_call(
        _copy_keys_body,
        grid=(B // BB,),
        in_specs=[pl.BlockSpec((BB, T, D), lambda b: (b, 0, 0))],
        out_specs=[
            pl.BlockSpec((BB, T + LP, D), lambda b: (b, 0, 0)),
            pl.BlockSpec((BB, D), lambda b: (b, 0)),
        ],
        out_shape=[
            jax.ShapeDtypeStruct((B, T + LP, D), jnp.float32),
            jax.ShapeDtypeStruct((B, D), jnp.float32),
        ],
    )(x_embed)


def _top4(s, iota, gbase):
    vs, gs = [], []
    for _ in range(K):
        v = jnp.max(s, axis=1, keepdims=True)
        a = jnp.min(jnp.where(s == v, iota, jnp.int32(2 ** 30)),
                    axis=1, keepdims=True)
        vs.append(v)
        gs.append(a + gbase)
        s = jnp.where(iota == a, NEG, s)
    return jnp.concatenate(vs, axis=1), jnp.concatenate(gs, axis=1)


def _sim_topk_body(xn_ref, pk_ref, sim_ref, tv_ref, ti_ref, idx_ref,
                   rv_ref, ri_ref):
    j = pl.program_id(0)
    xn = xn_ref[...]
    pk = pk_ref[...]
    sq = jnp.sum(pk * pk, axis=1, keepdims=True)
    pkn = pk * lax.rsqrt(jnp.maximum(sq, 1e-12))
    s = lax.dot_general(xn, pkn, (((1,), (1,)), ((), ())),
                        preferred_element_type=jnp.float32)
    sim_ref[...] = s

    iota = lax.broadcasted_iota(jnp.int32, (B, PB), 1)
    bv, bi = _top4(s, iota, j * PB)

    @pl.when(j == 0)
    def _():
        rv_ref[...] = bv
        ri_ref[...] = bi

    @pl.when(j > 0)
    def _():
        cv = jnp.concatenate([rv_ref[...], bv], axis=1)
        ci = jnp.concatenate([ri_ref[...], bi], axis=1)
        iota8 = lax.broadcasted_iota(jnp.int32, (B, 2 * K), 1)
        nvs, nis = [], []
        for _ in range(K):
            v = jnp.max(cv, axis=1, keepdims=True)
            a = jnp.min(jnp.where(cv == v, iota8, jnp.int32(2 ** 30)),
                        axis=1, keepdims=True)
            pick = jnp.sum(jnp.where(iota8 == a, ci, 0), axis=1,
                           keepdims=True)
            nvs.append(v)
            nis.append(pick)
            cv = jnp.where(iota8 == a, NEG, cv)
        rv_ref[...] = jnp.concatenate(nvs, axis=1)
        ri_ref[...] = jnp.concatenate(nis, axis=1)

    @pl.when(j == P // PB - 1)
    def _():
        tv_ref[...] = rv_ref[...]
        ti_ref[...] = ri_ref[...]
        idx_ref[...] = ri_ref[:, 0]


def _sim_topk(xn, prompt_key):
    return pl.pallas_call(
        _sim_topk_body,
        grid=(P // PB,),
        in_specs=[
            pl.BlockSpec((B, D), lambda j: (0, 0)),
            pl.BlockSpec((PB, D), lambda j: (j, 0)),
        ],
        out_specs=[
            pl.BlockSpec((B, PB), lambda j: (0, j)),
            pl.BlockSpec((B, K), lambda j: (0, 0)),
            pl.BlockSpec((B, K), lambda j: (0, 0)),
            pl.BlockSpec((B,), lambda j: (0,)),
        ],
        out_shape=[
            jax.ShapeDtypeStruct((B, P), jnp.float32),
            jax.ShapeDtypeStruct((B, K), jnp.float32),
            jax.ShapeDtypeStruct((B, K), jnp.int32),
            jax.ShapeDtypeStruct((B,), jnp.int32),
        ],
        scratch_shapes=[
            pltpu.VMEM((B, K), jnp.float32),
            pltpu.VMEM((B, K), jnp.int32),
        ],
    )(xn, prompt_key)


def _combine_body(idx_ref, p_ref, x_ref, big_ref, out_ref):
    del idx_ref, big_ref
    out_ref[:, 0:LP, :] = p_ref[...]
    out_ref[:, LP:LP + 3, :] = x_ref[:, 0:3, :]


def _gather_combine(idx, prompt, x_embed, out_big):
    grid_spec = pltpu.PrefetchScalarGridSpec(
        num_scalar_prefetch=1,
        grid=(B,),
        in_specs=[
            pl.BlockSpec((1, LP, D), lambda b, idx_ref: (idx_ref[b], 0, 0)),
            pl.BlockSpec((1, 8, D), lambda b, idx_ref: (b, 0, 0)),
            pl.BlockSpec(memory_space=pl.ANY),
        ],
        out_specs=pl.BlockSpec((1, 8, D), lambda b, idx_ref: (b, 0, 0)),
    )
    return pl.pallas_call(
        _combine_body,
        grid_spec=grid_spec,
        out_shape=jax.ShapeDtypeStruct((B, T + LP, D), jnp.float32),
        input_output_aliases={3: 0},
    )(idx, prompt, x_embed, out_big)


def kernel(x_embed, prompt, prompt_key):
    out_big, xn = _copy_and_keys(x_embed)
    sim, top_k_sim, top_k_idx, idx = _sim_topk(xn, prompt_key)
    outf = _gather_combine(idx, prompt, x_embed, out_big)
    return (sim, top_k_sim, top_k_idx, idx, outf)


# speedup vs baseline: 3.7646x; 2.7098x over previous
import jax
import jax.numpy as jnp
from jax import lax
from jax.experimental import pallas as pl
from jax.experimental.pallas import tpu as pltpu

B = 256
T = 196
D = 768
P = 8192
LP = 5
K = 4
PB = 1024
QB = 8
NEG = float("-inf")


def _copy_keys_body(x_ref, out_ref, xn_ref):
    c = pl.program_id(0)
    blk = x_ref[...]
    out_ref[...] = blk
    row = blk[0]

    @pl.when(c == 0)
    def _():
        xn_ref[...] = row

    @pl.when(c > 0)
    def _():
        xn_ref[...] += row

    @pl.when(c == T - 1)
    def _():
        m = xn_ref[...] * jnp.float32(1.0 / T)
        sq = jnp.sum(m * m, axis=1, keepdims=True)
        xn_ref[...] = m * lax.rsqrt(jnp.maximum(sq, 1e-12))


def _copy_and_keys(x_t):
    return pl.pallas_call(
        _copy_keys_body,
        grid=(T,),
        in_specs=[pl.BlockSpec((1, B, D), lambda c: (c, 0, 0))],
        out_specs=[
            pl.BlockSpec((1, B, D), lambda c: (c + LP, 0, 0)),
            pl.BlockSpec((B, D), lambda c: (0, 0)),
        ],
        out_shape=[
            jax.ShapeDtypeStruct((T + LP, B, D), jnp.float32),
            jax.ShapeDtypeStruct((B, D), jnp.float32),
        ],
    )(x_t)


def _top4(s, iota, gbase):
    vs, gs = [], []
    for _ in range(K):
        v = jnp.max(s, axis=1, keepdims=True)
        a = jnp.min(jnp.where(s == v, iota, jnp.int32(2 ** 30)),
                    axis=1, keepdims=True)
        vs.append(v)
        gs.append(a + gbase)
        s = jnp.where(iota == a, NEG, s)
    return jnp.concatenate(vs, axis=1), jnp.concatenate(gs, axis=1)


def _sim_topk_body(xn_ref, pk_ref, sim_ref, tv_ref, ti_ref, idx_ref,
                   rv_ref, ri_ref):
    j = pl.program_id(0)
    xn = xn_ref[...]
    pk = pk_ref[...]
    sq = jnp.sum(pk * pk, axis=1, keepdims=True)
    pkn = pk * lax.rsqrt(jnp.maximum(sq, 1e-12))
    s = lax.dot_general(xn, pkn, (((1,), (1,)), ((), ())),
                        preferred_element_type=jnp.float32)
    sim_ref[...] = s

    iota = lax.broadcasted_iota(jnp.int32, (B, PB), 1)
    bv, bi = _top4(s, iota, j * PB)

    @pl.when(j == 0)
    def _():
        rv_ref[...] = bv
        ri_ref[...] = bi

    @pl.when(j > 0)
    def _():
        cv = jnp.concatenate([rv_ref[...], bv], axis=1)
        ci = jnp.concatenate([ri_ref[...], bi], axis=1)
        iota8 = lax.broadcasted_iota(jnp.int32, (B, 2 * K), 1)
        nvs, nis = [], []
        for _ in range(K):
            v = jnp.max(cv, axis=1, keepdims=True)
            a = jnp.min(jnp.where(cv == v, iota8, jnp.int32(2 ** 30)),
                        axis=1, keepdims=True)
            pick = jnp.sum(jnp.where(iota8 == a, ci, 0), axis=1,
                           keepdims=True)
            nvs.append(v)
            nis.append(pick)
            cv = jnp.where(iota8 == a, NEG, cv)
        rv_ref[...] = jnp.concatenate(nvs, axis=1)
        ri_ref[...] = jnp.concatenate(nis, axis=1)

    @pl.when(j == P // PB - 1)
    def _():
        tv_ref[...] = rv_ref[...]
        ti_ref[...] = ri_ref[...]
        idx_ref[...] = ri_ref[:, 0]


def _sim_topk(xn, prompt_key):
    return pl.pallas_call(
        _sim_topk_body,
        grid=(P // PB,),
        in_specs=[
            pl.BlockSpec((B, D), lambda j: (0, 0)),
            pl.BlockSpec((PB, D), lambda j: (j, 0)),
        ],
        out_specs=[
            pl.BlockSpec((B, PB), lambda j: (0, j)),
            pl.BlockSpec((B, K), lambda j: (0, 0)),
            pl.BlockSpec((B, K), lambda j: (0, 0)),
            pl.BlockSpec((B,), lambda j: (0,)),
        ],
        out_shape=[
            jax.ShapeDtypeStruct((B, P), jnp.float32),
            jax.ShapeDtypeStruct((B, K), jnp.float32),
            jax.ShapeDtypeStruct((B, K), jnp.int32),
            jax.ShapeDtypeStruct((B,), jnp.int32),
        ],
        scratch_shapes=[
            pltpu.VMEM((B, K), jnp.float32),
            pltpu.VMEM((B, K), jnp.int32),
        ],
    )(xn, prompt_key)


def _combine_body(idx_ref, *refs):
    b = pl.program_id(0)
    p_refs = refs[:QB]
    out_ref = refs[QB + 1]
    for i in range(QB):
        r = idx_ref[b * QB + i] % QB
        out_ref[0:LP, pl.ds(i, 1), :] = p_refs[i][:, pl.ds(r, 1), :]


def _gather_combine(idx, p_t, out_big):
    grid_spec = pltpu.PrefetchScalarGridSpec(
        num_scalar_prefetch=1,
        grid=(B // QB,),
        in_specs=[
            pl.BlockSpec((LP, QB, D),
                         lambda b, idx_ref, i=i: (0, idx_ref[b * QB + i] // QB, 0))
            for i in range(QB)
        ] + [
            pl.BlockSpec(memory_space=pl.ANY),
        ],
        out_specs=pl.BlockSpec((LP, QB, D), lambda b, idx_ref: (0, b, 0)),
    )
    return pl.pallas_call(
        _combine_body,
        grid_spec=grid_spec,
        out_shape=jax.ShapeDtypeStruct((T + LP, B, D), jnp.float32),
        input_output_aliases={QB + 1: 0},
    )(idx, *([p_t] * QB), out_big)


def kernel(x_embed, prompt, prompt_key):
    x_t = jnp.transpose(x_embed, (1, 0, 2))
    p_t = jnp.transpose(prompt, (1, 0, 2))
    out_big, xn = _copy_and_keys(x_t)
    sim, top_k_sim, top_k_idx, idx = _sim_topk(xn, prompt_key)
    out_t = _gather_combine(idx, p_t, out_big)
    return (sim, top_k_sim, top_k_idx, idx, jnp.transpose(out_t, (1, 0, 2)))
